# baseline (speedup 1.0000x reference)
import jax
import jax.numpy as jnp
from jax import lax
from jax.experimental import pallas as pl
from jax.experimental.pallas import tpu as pltpu
from jax.experimental.pallas import tpu_sc as plsc

N_NODES = 10000
MAX_DEG = 32
D_FEAT = 128
_NW = 32
_C = 312
_T = 8
_NT = _C // _T
_NB = 3
_TAIL0 = _NW * _C
_INV = 1.0 / MAX_DEG


def _reduce_tile(buf, obuf):
    @plsc.parallel_loop(0, _T)
    def _node(n):
        for c in range(D_FEAT // 16):
            sl = pl.ds(c * 16, 16)
            acc = buf[n, 0, sl]
            for k in range(1, MAX_DEG):
                acc = acc + buf[n, k, sl]
            obuf[n, sl] = acc * _INV


def _sc_body(mail, out, buf0, buf1, buf2, ob0, ob1, ob2,
             sem0, sem1, sem2, osem0, osem1, osem2):
    w = lax.axis_index("s") * 2 + lax.axis_index("c")
    base = w * _C
    bufs = (buf0, buf1, buf2)
    obs = (ob0, ob1, ob2)
    sems = (sem0, sem1, sem2)
    osems = (osem0, osem1, osem2)

    for b in range(_NB):
        pltpu.async_copy(mail.at[pl.ds(base + b * _T, _T)], bufs[b], sems[b])

    def group(i, carry):
        t0 = i * _NB
        for b in range(_NB):
            t = t0 + b
            node0 = base + t * _T
            pltpu.make_async_copy(mail.at[pl.ds(node0, _T)], bufs[b], sems[b]).wait()

            @pl.when(i >= 1)
            def _():
                pltpu.make_async_copy(obs[b], out.at[pl.ds(node0, _T)], osems[b]).wait()

            _reduce_tile(bufs[b], obs[b])
            pltpu.async_copy(obs[b], out.at[pl.ds(node0, _T)], osems[b])

            @pl.when(t + _NB < _NT)
            def _():
                pltpu.async_copy(
                    mail.at[pl.ds(node0 + _NB * _T, _T)], bufs[b], sems[b])
        return carry

    lax.fori_loop(0, _NT // _NB, group, 0)

    for b in range(_NB):
        pltpu.make_async_copy(obs[b], out.at[pl.ds(base, _T)], osems[b]).wait()

    @pl.when(w < 2)
    def _():
        node0 = _TAIL0 + w * _T
        pltpu.sync_copy(mail.at[pl.ds(node0, _T)], buf0)
        _reduce_tile(buf0, ob0)
        pltpu.sync_copy(ob0, out.at[pl.ds(node0, _T)])


def kernel(mailbox_m):
    mesh = plsc.VectorSubcoreMesh(core_axis_name="c", subcore_axis_name="s")
    f = pl.kernel(
        _sc_body,
        out_type=jax.ShapeDtypeStruct((N_NODES, D_FEAT), jnp.float32),
        mesh=mesh,
        scratch_types=[
            pltpu.VMEM((_T, MAX_DEG, D_FEAT), jnp.float32),
            pltpu.VMEM((_T, MAX_DEG, D_FEAT), jnp.float32),
            pltpu.VMEM((_T, MAX_DEG, D_FEAT), jnp.float32),
            pltpu.VMEM((_T, D_FEAT), jnp.float32),
            pltpu.VMEM((_T, D_FEAT), jnp.float32),
            pltpu.VMEM((_T, D_FEAT), jnp.float32),
            pltpu.SemaphoreType.DMA,
            pltpu.SemaphoreType.DMA,
            pltpu.SemaphoreType.DMA,
            pltpu.SemaphoreType.DMA,
            pltpu.SemaphoreType.DMA,
            pltpu.SemaphoreType.DMA,
        ],
    )
    return f(mailbox_m)
